# Initial kernel scaffold; baseline (speedup 1.0000x reference)
import functools

import jax
import jax.numpy as jnp
from jax import lax
from jax.experimental import pallas as pl
from jax.experimental.pallas import tpu as pltpu
from jax.experimental.pallas import tpu_sc as plsc

N_FINE = 10000
N_COARSE = 2500
E = 320000
W = 128
F32 = jnp.float32


BN = 2000
BE = 2000


def _dot(a, b):
    return jnp.dot(a, b, preferred_element_type=F32)


def _unpool_body(v, w1, b1, w2, b2, o):
    h = jnp.maximum(_dot(v[...], w1[...]) + b1[...], 0.0)
    o[...] = _dot(h, w2[...]) + b2[...]


def _tc_unpool(v, w1, b1, w2, b2):
    return pl.pallas_call(
        _unpool_body,
        out_shape=jax.ShapeDtypeStruct((N_COARSE, W), F32),
    )(v, w1, b1, w2, b2)


def _row_spec(nrows):
    return pl.BlockSpec((nrows, W), lambda i: (i, 0))


def _full_spec(shape):
    return pl.BlockSpec(shape, lambda i: (0,) * len(shape))


def _nodeprep_body(vf, cs, ws, wd, b1, v_o, p_o, q_o):
    val = vf[...] + cs[...]
    v_o[...] = val
    p_o[...] = _dot(val, ws[...]) + b1[...]
    q_o[...] = _dot(val, wd[...])


def _tc_nodeprep(vf_pad, c_skip, ws, wd, b1):
    return pl.pallas_call(
        _nodeprep_body,
        grid=(N_FINE // BN,),
        in_specs=[
            _row_spec(BN),
            _row_spec(BN),
            _full_spec((W, W)),
            _full_spec((W, W)),
            _full_spec((1, W)),
        ],
        out_specs=[_row_spec(BN), _row_spec(BN), _row_spec(BN)],
        out_shape=[jax.ShapeDtypeStruct((N_FINE, W), F32)] * 3,
    )(vf_pad, c_skip, ws, wd, b1)


def _edge_body(e, g1, g2, w1e, w2, b2, o):
    h = jnp.maximum(_dot(e[...], w1e[...]) + g1[...] + g2[...], 0.0)
    o[...] = e[...] + _dot(h, w2[...]) + b2[...]


def _tc_edge(e, g1, g2, w1e, w2, b2):
    return pl.pallas_call(
        _edge_body,
        grid=(E // BE,),
        in_specs=[
            _row_spec(BE),
            _row_spec(BE),
            _row_spec(BE),
            _full_spec((W, W)),
            _full_spec((W, W)),
            _full_spec((1, W)),
        ],
        out_specs=_row_spec(BE),
        out_shape=jax.ShapeDtypeStruct((E, W), F32),
    )(e, g1, g2, w1e, w2, b2)


def _node_mid_body(v, p0, p1, w1a, w1b, b1, w2, b2, ws, wd, eb, v_o, p_o, q_o):
    agg = p0[...] + p1[...]
    h = jnp.maximum(_dot(v[...], w1a[...]) + _dot(agg, w1b[...]) + b1[...], 0.0)
    vn = v[...] + _dot(h, w2[...]) + b2[...]
    v_o[...] = vn
    p_o[...] = _dot(vn, ws[...]) + eb[...]
    q_o[...] = _dot(vn, wd[...])


def _tc_node_mid(v, parts, w1a, w1b, b1, w2, b2, ws, wd, eb):
    nb = N_FINE // BN
    return pl.pallas_call(
        _node_mid_body,
        grid=(nb,),
        in_specs=[
            _row_spec(BN),
            _row_spec(BN),
            pl.BlockSpec((BN, W), lambda i: (i + nb, 0)),
            _full_spec((W, W)),
            _full_spec((W, W)),
            _full_spec((1, W)),
            _full_spec((W, W)),
            _full_spec((1, W)),
            _full_spec((W, W)),
            _full_spec((W, W)),
            _full_spec((1, W)),
        ],
        out_specs=[_row_spec(BN), _row_spec(BN), _row_spec(BN)],
        out_shape=[jax.ShapeDtypeStruct((N_FINE, W), F32)] * 3,
    )(v, parts, parts, w1a, w1b, b1, w2, b2, ws, wd, eb)


def _node_fin_body(v, p0, p1, w1a, w1b, b1, w2, b2, v_o):
    agg = p0[...] + p1[...]
    h = jnp.maximum(_dot(v[...], w1a[...]) + _dot(agg, w1b[...]) + b1[...], 0.0)
    v_o[...] = v[...] + _dot(h, w2[...]) + b2[...]


def _tc_node_fin(v, parts, w1a, w1b, b1, w2, b2):
    nb = N_FINE // BN
    return pl.pallas_call(
        _node_fin_body,
        grid=(nb,),
        in_specs=[
            _row_spec(BN),
            _row_spec(BN),
            pl.BlockSpec((BN, W), lambda i: (i + nb, 0)),
            _full_spec((W, W)),
            _full_spec((W, W)),
            _full_spec((1, W)),
            _full_spec((W, W)),
            _full_spec((1, W)),
        ],
        out_specs=_row_spec(BN),
        out_shape=jax.ShapeDtypeStruct((N_FINE, W), F32),
    )(v, parts, parts, w1a, w1b, b1, w2, b2)


_MESH = plsc.VectorSubcoreMesh(core_axis_name="c", subcore_axis_name="s")
NTILE = 32
CH = 80


def _sc_gather1(table, idx, n_idx):
    per = n_idx // NTILE
    nch = per // CH

    @functools.partial(
        pl.kernel,
        out_type=jax.ShapeDtypeStruct((n_idx, W), F32),
        mesh=_MESH,
        scratch_types=[
            pltpu.VMEM((CH,), jnp.int32),
            pltpu.VMEM((CH, W), F32),
            pltpu.SemaphoreType.DMA,
        ],
    )
    def k(t_hbm, i_hbm, o_hbm, iv, rv, sem):
        wid = lax.axis_index("c") * 16 + lax.axis_index("s")
        base = wid * per

        @pl.loop(0, nch)
        def _(c):
            off = base + c * CH
            pltpu.sync_copy(i_hbm.at[pl.ds(off, CH)], iv)
            pltpu.async_copy(t_hbm.at[iv], rv, sem).wait()
            pltpu.sync_copy(rv, o_hbm.at[pl.ds(off, CH)])

    return k(table, idx)


def _sc_gather2(p, q, src, dst):
    per = E // NTILE
    nch = per // CH

    @functools.partial(
        pl.kernel,
        out_type=(
            jax.ShapeDtypeStruct((E, W), F32),
            jax.ShapeDtypeStruct((E, W), F32),
        ),
        mesh=_MESH,
        scratch_types=[
            pltpu.VMEM((CH,), jnp.int32),
            pltpu.VMEM((CH,), jnp.int32),
            pltpu.VMEM((CH, W), F32),
            pltpu.VMEM((CH, W), F32),
            pltpu.SemaphoreType.DMA,
            pltpu.SemaphoreType.DMA,
        ],
    )
    def k(p_hbm, q_hbm, s_hbm, d_hbm, g1_hbm, g2_hbm, si, di, r1, r2, m1, m2):
        wid = lax.axis_index("c") * 16 + lax.axis_index("s")
        base = wid * per

        @pl.loop(0, nch)
        def _(c):
            off = base + c * CH
            pltpu.sync_copy(s_hbm.at[pl.ds(off, CH)], si)
            pltpu.sync_copy(d_hbm.at[pl.ds(off, CH)], di)
            cp1 = pltpu.async_copy(p_hbm.at[si], r1, m1)
            cp2 = pltpu.async_copy(q_hbm.at[di], r2, m2)
            cp1.wait()
            cp2.wait()
            pltpu.sync_copy(r1, g1_hbm.at[pl.ds(off, CH)])
            pltpu.sync_copy(r2, g2_hbm.at[pl.ds(off, CH)])

    return k(p, q, src, dst)


def _sc_scatter_add(e, dst, zeros):
    per = E // NTILE
    nch = per // CH
    zper = N_FINE // 16

    @functools.partial(
        pl.kernel,
        out_type=jax.ShapeDtypeStruct((2 * N_FINE, W), F32),
        mesh=_MESH,
        scratch_types=[
            pltpu.VMEM((CH,), jnp.int32),
            pltpu.VMEM((CH, W), F32),
            pltpu.VMEM_SHARED((N_FINE, W), F32),
        ],
    )
    def k(e_hbm, d_hbm, z_hbm, o_hbm, di, rv, agg):
        cid = lax.axis_index("c")
        sid = lax.axis_index("s")
        wid = cid * 16 + sid
        pltpu.sync_copy(
            z_hbm.at[pl.ds(sid * zper, zper)], agg.at[pl.ds(sid * zper, zper)]
        )
        plsc.subcore_barrier()
        base = wid * per

        @pl.loop(0, nch)
        def _(c):
            off = base + c * CH
            pltpu.sync_copy(d_hbm.at[pl.ds(off, CH)], di)
            pltpu.sync_copy(e_hbm.at[pl.ds(off, CH)], rv)
            pltpu.sync_copy(rv, agg.at[di], add=True)

        plsc.subcore_barrier()
        pltpu.sync_copy(
            agg.at[pl.ds(sid * zper, zper)],
            o_hbm.at[pl.ds(cid * N_FINE + sid * zper, zper)],
        )

    return k(e, dst, zeros)


def _r(b):
    return b.reshape(1, W)


def kernel(edge_index, v, c_skip, e_skip, edge_index_skip, batch_skip,
           up_W1, up_b1, up_W2, up_b2,
           eW1, eb1, eW2, eb2, nW1, nb1, nW2, nb2):
    del batch_skip
    src = edge_index[0]
    dst = edge_index[1]
    n_pad = 10240
    eis_pad = jnp.concatenate(
        [edge_index_skip, jnp.zeros((n_pad - N_FINE,), jnp.int32)]
    )
    zeros = jnp.zeros((N_FINE, W), F32)

    h = _tc_unpool(v, up_W1, _r(up_b1), up_W2, _r(up_b2))
    vf_pad = _sc_gather1(h, eis_pad, n_pad)
    v_cur, p, q = _tc_nodeprep(
        vf_pad, c_skip, eW1[0, W:2 * W], eW1[0, 2 * W:], _r(eb1[0])
    )

    e = e_skip
    for d in range(2):
        g1, g2 = _sc_gather2(p, q, src, dst)
        e = _tc_edge(e, g1, g2, eW1[d, :W], eW2[d], _r(eb2[d]))
        parts = _sc_scatter_add(e, dst, zeros)
        if d == 0:
            v_cur, p, q = _tc_node_mid(
                v_cur, parts, nW1[d, :W], nW1[d, W:], _r(nb1[d]),
                nW2[d], _r(nb2[d]),
                eW1[1, W:2 * W], eW1[1, 2 * W:], _r(eb1[1]),
            )
        else:
            v_cur = _tc_node_fin(
                v_cur, parts, nW1[d, :W], nW1[d, W:], _r(nb1[d]),
                nW2[d], _r(nb2[d]),
            )
    return v_cur


# speedup vs baseline: 2.7137x; 2.7137x over previous
import functools

import jax
import jax.numpy as jnp
from jax import lax
from jax.experimental import pallas as pl
from jax.experimental.pallas import tpu as pltpu
from jax.experimental.pallas import tpu_sc as plsc

N_FINE = 10000
N_COARSE = 2500
E = 320000
W = 128
F32 = jnp.float32


BN = 2000
BE = 2000


def _dot(a, b):
    return jnp.dot(a, b, preferred_element_type=F32)


def _unpool_body(v, w1, b1, w2, b2, o):
    h = jnp.maximum(_dot(v[...], w1[...]) + b1[...], 0.0)
    o[...] = _dot(h, w2[...]) + b2[...]


def _tc_unpool(v, w1, b1, w2, b2):
    return pl.pallas_call(
        _unpool_body,
        out_shape=jax.ShapeDtypeStruct((N_COARSE, W), F32),
    )(v, w1, b1, w2, b2)


def _row_spec(nrows):
    return pl.BlockSpec((nrows, W), lambda i: (i, 0))


def _full_spec(shape):
    return pl.BlockSpec(shape, lambda i: (0,) * len(shape))


def _nodeprep_body(vf, cs, ws, wd, b1, v_o, p_o, q_o):
    val = vf[...] + cs[...]
    v_o[...] = val
    p_o[...] = _dot(val, ws[...]) + b1[...]
    q_o[...] = _dot(val, wd[...])


def _tc_nodeprep(vf_pad, c_skip, ws, wd, b1):
    return pl.pallas_call(
        _nodeprep_body,
        grid=(N_FINE // BN,),
        in_specs=[
            _row_spec(BN),
            _row_spec(BN),
            _full_spec((W, W)),
            _full_spec((W, W)),
            _full_spec((1, W)),
        ],
        out_specs=[_row_spec(BN), _row_spec(BN), _row_spec(BN)],
        out_shape=[jax.ShapeDtypeStruct((N_FINE, W), F32)] * 3,
    )(vf_pad, c_skip, ws, wd, b1)


def _edge_body(e, g1, g2, w1e, w2, b2, o):
    h = jnp.maximum(_dot(e[...], w1e[...]) + g1[...] + g2[...], 0.0)
    o[...] = e[...] + _dot(h, w2[...]) + b2[...]


def _tc_edge(e, g1, g2, w1e, w2, b2):
    return pl.pallas_call(
        _edge_body,
        grid=(E // BE,),
        in_specs=[
            _row_spec(BE),
            _row_spec(BE),
            _row_spec(BE),
            _full_spec((W, W)),
            _full_spec((W, W)),
            _full_spec((1, W)),
        ],
        out_specs=_row_spec(BE),
        out_shape=jax.ShapeDtypeStruct((E, W), F32),
    )(e, g1, g2, w1e, w2, b2)


def _node_mid_body(v, p0, p1, w1a, w1b, b1, w2, b2, ws, wd, eb, v_o, p_o, q_o):
    agg = p0[...] + p1[...]
    h = jnp.maximum(_dot(v[...], w1a[...]) + _dot(agg, w1b[...]) + b1[...], 0.0)
    vn = v[...] + _dot(h, w2[...]) + b2[...]
    v_o[...] = vn
    p_o[...] = _dot(vn, ws[...]) + eb[...]
    q_o[...] = _dot(vn, wd[...])


def _tc_node_mid(v, p0, p1, w1a, w1b, b1, w2, b2, ws, wd, eb):
    nb = N_FINE // BN
    return pl.pallas_call(
        _node_mid_body,
        grid=(nb,),
        in_specs=[
            _row_spec(BN),
            _row_spec(BN),
            _row_spec(BN),
            _full_spec((W, W)),
            _full_spec((W, W)),
            _full_spec((1, W)),
            _full_spec((W, W)),
            _full_spec((1, W)),
            _full_spec((W, W)),
            _full_spec((W, W)),
            _full_spec((1, W)),
        ],
        out_specs=[_row_spec(BN), _row_spec(BN), _row_spec(BN)],
        out_shape=[jax.ShapeDtypeStruct((N_FINE, W), F32)] * 3,
    )(v, p0, p1, w1a, w1b, b1, w2, b2, ws, wd, eb)


def _node_fin_body(v, p0, p1, w1a, w1b, b1, w2, b2, v_o):
    agg = p0[...] + p1[...]
    h = jnp.maximum(_dot(v[...], w1a[...]) + _dot(agg, w1b[...]) + b1[...], 0.0)
    v_o[...] = v[...] + _dot(h, w2[...]) + b2[...]


def _tc_node_fin(v, p0, p1, w1a, w1b, b1, w2, b2):
    nb = N_FINE // BN
    return pl.pallas_call(
        _node_fin_body,
        grid=(nb,),
        in_specs=[
            _row_spec(BN),
            _row_spec(BN),
            _row_spec(BN),
            _full_spec((W, W)),
            _full_spec((W, W)),
            _full_spec((1, W)),
            _full_spec((W, W)),
            _full_spec((1, W)),
        ],
        out_specs=_row_spec(BN),
        out_shape=jax.ShapeDtypeStruct((N_FINE, W), F32),
    )(v, p0, p1, w1a, w1b, b1, w2, b2)


@functools.cache
def _sc_mesh():
    return plsc.VectorSubcoreMesh(core_axis_name="c", subcore_axis_name="s")
NTILE = 32
CH = 80


def _sc_gather1(table, idx, n_idx):
    per = n_idx // NTILE
    nch = per // CH

    @functools.partial(
        pl.kernel,
        out_type=jax.ShapeDtypeStruct((n_idx, W), F32),
        mesh=_sc_mesh(),
        scratch_types=[
            pltpu.VMEM((CH,), jnp.int32),
            pltpu.VMEM((CH, W), F32),
            pltpu.SemaphoreType.DMA,
        ],
    )
    def k(t_hbm, i_hbm, o_hbm, iv, rv, sem):
        wid = lax.axis_index("c") * 16 + lax.axis_index("s")
        base = wid * per

        @pl.loop(0, nch)
        def _(c):
            off = base + c * CH
            pltpu.sync_copy(i_hbm.at[pl.ds(off, CH)], iv)
            pltpu.async_copy(t_hbm.at[iv], rv, sem).wait()
            pltpu.sync_copy(rv, o_hbm.at[pl.ds(off, CH)])

    return k(table, idx)


def _sc_gather2(p, q, src, dst):
    per = E // NTILE
    nch = per // CH

    @functools.partial(
        pl.kernel,
        out_type=(
            jax.ShapeDtypeStruct((E, W), F32),
            jax.ShapeDtypeStruct((E, W), F32),
        ),
        mesh=_sc_mesh(),
        scratch_types=[
            pltpu.VMEM((CH,), jnp.int32),
            pltpu.VMEM((CH,), jnp.int32),
            pltpu.VMEM((CH, W), F32),
            pltpu.VMEM((CH, W), F32),
            pltpu.SemaphoreType.DMA,
            pltpu.SemaphoreType.DMA,
        ],
    )
    def k(p_hbm, q_hbm, s_hbm, d_hbm, g1_hbm, g2_hbm, si, di, r1, r2, m1, m2):
        wid = lax.axis_index("c") * 16 + lax.axis_index("s")
        base = wid * per

        @pl.loop(0, nch)
        def _(c):
            off = base + c * CH
            pltpu.sync_copy(s_hbm.at[pl.ds(off, CH)], si)
            pltpu.sync_copy(d_hbm.at[pl.ds(off, CH)], di)
            cp1 = pltpu.async_copy(p_hbm.at[si], r1, m1)
            cp2 = pltpu.async_copy(q_hbm.at[di], r2, m2)
            cp1.wait()
            cp2.wait()
            pltpu.sync_copy(r1, g1_hbm.at[pl.ds(off, CH)])
            pltpu.sync_copy(r2, g2_hbm.at[pl.ds(off, CH)])

    return k(p, q, src, dst)


AGGP = 10240


def _sc_scatter_add(e, dst, zeros):
    per = E // NTILE
    nch = per // CH
    zper = AGGP // 16

    @functools.partial(
        pl.kernel,
        out_type=jax.ShapeDtypeStruct((2 * AGGP, W), F32),
        mesh=_sc_mesh(),
        scratch_types=[
            pltpu.VMEM((CH,), jnp.int32),
            pltpu.VMEM((CH, W), F32),
            pltpu.VMEM_SHARED((AGGP, W), F32),
        ],
    )
    def k(e_hbm, d_hbm, z_hbm, o_hbm, di, rv, agg):
        cid = lax.axis_index("c")
        sid = lax.axis_index("s")
        wid = cid * 16 + sid
        pltpu.sync_copy(
            z_hbm.at[pl.ds(sid * zper, zper)], agg.at[pl.ds(sid * zper, zper)]
        )
        plsc.subcore_barrier()
        base = wid * per

        @pl.loop(0, nch)
        def _(c):
            off = base + c * CH
            pltpu.sync_copy(d_hbm.at[pl.ds(off, CH)], di)
            pltpu.sync_copy(e_hbm.at[pl.ds(off, CH)], rv)
            pltpu.sync_copy(rv, agg.at[di], add=True)

        plsc.subcore_barrier()
        pltpu.sync_copy(
            agg.at[pl.ds(sid * zper, zper)],
            o_hbm.at[pl.ds(cid * AGGP + sid * zper, zper)],
        )

    return k(e, dst, zeros)


def _r(b):
    return b.reshape(1, W)


def kernel(edge_index, v, c_skip, e_skip, edge_index_skip, batch_skip,
           up_W1, up_b1, up_W2, up_b2,
           eW1, eb1, eW2, eb2, nW1, nb1, nW2, nb2):
    del batch_skip
    src = edge_index[0]
    dst = edge_index[1]
    n_pad = 10240
    eis_pad = jnp.concatenate(
        [edge_index_skip, jnp.zeros((n_pad - N_FINE,), jnp.int32)]
    )
    zeros = jnp.zeros((AGGP, W), F32)

    h = _tc_unpool(v, up_W1, _r(up_b1), up_W2, _r(up_b2))
    vf_pad = _sc_gather1(h, eis_pad, n_pad)
    v_cur, p, q = _tc_nodeprep(
        vf_pad, c_skip, eW1[0, W:2 * W], eW1[0, 2 * W:], _r(eb1[0])
    )

    e = e_skip
    for d in range(2):
        g1, g2 = _sc_gather2(p, q, src, dst)
        e = _tc_edge(e, g1, g2, eW1[d, :W], eW2[d], _r(eb2[d]))
        parts = _sc_scatter_add(e, dst, zeros)
        p0 = parts[:N_FINE]
        p1 = parts[AGGP:AGGP + N_FINE]
        if d == 0:
            v_cur, p, q = _tc_node_mid(
                v_cur, p0, p1, nW1[d, :W], nW1[d, W:], _r(nb1[d]),
                nW2[d], _r(nb2[d]),
                eW1[1, W:2 * W], eW1[1, 2 * W:], _r(eb1[1]),
            )
        else:
            v_cur = _tc_node_fin(
                v_cur, p0, p1, nW1[d, :W], nW1[d, W:], _r(nb1[d]),
                nW2[d], _r(nb2[d]),
            )
    return v_cur


# speedup vs baseline: 3.6585x; 1.3482x over previous
import functools

import jax
import jax.numpy as jnp
from jax import lax
from jax.experimental import pallas as pl
from jax.experimental.pallas import tpu as pltpu
from jax.experimental.pallas import tpu_sc as plsc

N_FINE = 10000
N_COARSE = 2500
E = 320000
W = 128
F32 = jnp.float32


BN = 2000
BE = 2000


def _dot(a, b):
    return jnp.dot(a, b, preferred_element_type=F32)


def _unpool_body(v, w1, b1, w2, b2, o):
    h = jnp.maximum(_dot(v[...], w1[...]) + b1[...], 0.0)
    o[...] = _dot(h, w2[...]) + b2[...]


def _tc_unpool(v, w1, b1, w2, b2):
    return pl.pallas_call(
        _unpool_body,
        out_shape=jax.ShapeDtypeStruct((N_COARSE, W), F32),
    )(v, w1, b1, w2, b2)


def _row_spec(nrows):
    return pl.BlockSpec((nrows, W), lambda i: (i, 0))


def _full_spec(shape):
    return pl.BlockSpec(shape, lambda i: (0,) * len(shape))


def _nodeprep_body(vf, cs, ws, wd, b1, v_o, p_o, q_o):
    val = vf[...] + cs[...]
    v_o[...] = val
    p_o[...] = _dot(val, ws[...]) + b1[...]
    q_o[...] = _dot(val, wd[...])


def _tc_nodeprep(vf_pad, c_skip, ws, wd, b1):
    return pl.pallas_call(
        _nodeprep_body,
        grid=(N_FINE // BN,),
        in_specs=[
            _row_spec(BN),
            _row_spec(BN),
            _full_spec((W, W)),
            _full_spec((W, W)),
            _full_spec((1, W)),
        ],
        out_specs=[_row_spec(BN), _row_spec(BN), _row_spec(BN)],
        out_shape=[jax.ShapeDtypeStruct((N_FINE, W), F32)] * 3,
    )(vf_pad, c_skip, ws, wd, b1)


def _edge_body(e, g1, g2, w1e, w2, b2, o):
    h = jnp.maximum(_dot(e[...], w1e[...]) + g1[...] + g2[...], 0.0)
    o[...] = e[...] + _dot(h, w2[...]) + b2[...]


def _tc_edge(e, g1, g2, w1e, w2, b2):
    return pl.pallas_call(
        _edge_body,
        grid=(E // BE,),
        in_specs=[
            _row_spec(BE),
            _row_spec(BE),
            _row_spec(BE),
            _full_spec((W, W)),
            _full_spec((W, W)),
            _full_spec((1, W)),
        ],
        out_specs=_row_spec(BE),
        out_shape=jax.ShapeDtypeStruct((E, W), F32),
    )(e, g1, g2, w1e, w2, b2)


def _node_mid_body(v, p0, p1, w1a, w1b, b1, w2, b2, ws, wd, eb, v_o, p_o, q_o):
    agg = p0[...] + p1[...]
    h = jnp.maximum(_dot(v[...], w1a[...]) + _dot(agg, w1b[...]) + b1[...], 0.0)
    vn = v[...] + _dot(h, w2[...]) + b2[...]
    v_o[...] = vn
    p_o[...] = _dot(vn, ws[...]) + eb[...]
    q_o[...] = _dot(vn, wd[...])


def _tc_node_mid(v, p0, p1, w1a, w1b, b1, w2, b2, ws, wd, eb):
    nb = N_FINE // BN
    return pl.pallas_call(
        _node_mid_body,
        grid=(nb,),
        in_specs=[
            _row_spec(BN),
            _row_spec(BN),
            _row_spec(BN),
            _full_spec((W, W)),
            _full_spec((W, W)),
            _full_spec((1, W)),
            _full_spec((W, W)),
            _full_spec((1, W)),
            _full_spec((W, W)),
            _full_spec((W, W)),
            _full_spec((1, W)),
        ],
        out_specs=[_row_spec(BN), _row_spec(BN), _row_spec(BN)],
        out_shape=[jax.ShapeDtypeStruct((N_FINE, W), F32)] * 3,
    )(v, p0, p1, w1a, w1b, b1, w2, b2, ws, wd, eb)


def _node_fin_body(v, p0, p1, w1a, w1b, b1, w2, b2, v_o):
    agg = p0[...] + p1[...]
    h = jnp.maximum(_dot(v[...], w1a[...]) + _dot(agg, w1b[...]) + b1[...], 0.0)
    v_o[...] = v[...] + _dot(h, w2[...]) + b2[...]


def _tc_node_fin(v, p0, p1, w1a, w1b, b1, w2, b2):
    nb = N_FINE // BN
    return pl.pallas_call(
        _node_fin_body,
        grid=(nb,),
        in_specs=[
            _row_spec(BN),
            _row_spec(BN),
            _row_spec(BN),
            _full_spec((W, W)),
            _full_spec((W, W)),
            _full_spec((1, W)),
            _full_spec((W, W)),
            _full_spec((1, W)),
        ],
        out_specs=_row_spec(BN),
        out_shape=jax.ShapeDtypeStruct((N_FINE, W), F32),
    )(v, p0, p1, w1a, w1b, b1, w2, b2)


@functools.cache
def _sc_mesh():
    return plsc.VectorSubcoreMesh(core_axis_name="c", subcore_axis_name="s")
NTILE = 32
CH = 80


def _pipe2(nch, issue, process):
    issue(0, 0)
    issue(1, 1)
    if nch % 2 == 0:
        @pl.loop(0, (nch - 2) // 2)
        def _(j):
            process(2 * j, 0)
            issue(2 * j + 2, 0)
            process(2 * j + 1, 1)
            issue(2 * j + 3, 1)
        process(nch - 2, 0)
        process(nch - 1, 1)
    else:
        @pl.loop(0, (nch - 3) // 2)
        def _(j):
            process(2 * j, 0)
            issue(2 * j + 2, 0)
            process(2 * j + 1, 1)
            issue(2 * j + 3, 1)
        process(nch - 3, 0)
        issue(nch - 1, 0)
        process(nch - 2, 1)
        process(nch - 1, 0)


def _sc_gather1(table, idx, n_idx):
    per = n_idx // NTILE
    nch = per // CH

    @functools.partial(
        pl.kernel,
        out_type=jax.ShapeDtypeStruct((n_idx, W), F32),
        mesh=_sc_mesh(),
        scratch_types=[
            pltpu.VMEM((2, CH), jnp.int32),
            pltpu.VMEM((2, CH, W), F32),
            pltpu.SemaphoreType.DMA,
            pltpu.SemaphoreType.DMA,
        ],
    )
    def k(t_hbm, i_hbm, o_hbm, iv, rv, s0, s1):
        wid = lax.axis_index("c") * 16 + lax.axis_index("s")
        base = wid * per
        sems = (s0, s1)

        def issue(c, b):
            off = base + c * CH
            pltpu.sync_copy(i_hbm.at[pl.ds(off, CH)], iv.at[b])
            pltpu.async_copy(t_hbm.at[iv.at[b]], rv.at[b], sems[b])

        def process(c, b):
            off = base + c * CH
            pltpu.make_async_copy(t_hbm.at[iv.at[b]], rv.at[b], sems[b]).wait()
            pltpu.sync_copy(rv.at[b], o_hbm.at[pl.ds(off, CH)])

        _pipe2(nch, issue, process)

    return k(table, idx)


def _sc_gather2(p, q, src, dst):
    per = E // NTILE
    nch = per // CH

    @functools.partial(
        pl.kernel,
        out_type=(
            jax.ShapeDtypeStruct((E, W), F32),
            jax.ShapeDtypeStruct((E, W), F32),
        ),
        mesh=_sc_mesh(),
        scratch_types=[
            pltpu.VMEM((2, CH), jnp.int32),
            pltpu.VMEM((2, CH), jnp.int32),
            pltpu.VMEM((2, CH, W), F32),
            pltpu.VMEM((2, CH, W), F32),
            pltpu.SemaphoreType.DMA,
            pltpu.SemaphoreType.DMA,
            pltpu.SemaphoreType.DMA,
            pltpu.SemaphoreType.DMA,
        ],
    )
    def k(p_hbm, q_hbm, s_hbm, d_hbm, g1_hbm, g2_hbm,
          si, di, r1, r2, ma0, ma1, mb0, mb1):
        wid = lax.axis_index("c") * 16 + lax.axis_index("s")
        base = wid * per
        sa = (ma0, ma1)
        sb = (mb0, mb1)

        def issue(c, b):
            off = base + c * CH
            pltpu.sync_copy(s_hbm.at[pl.ds(off, CH)], si.at[b])
            pltpu.sync_copy(d_hbm.at[pl.ds(off, CH)], di.at[b])
            pltpu.async_copy(p_hbm.at[si.at[b]], r1.at[b], sa[b])
            pltpu.async_copy(q_hbm.at[di.at[b]], r2.at[b], sb[b])

        def process(c, b):
            off = base + c * CH
            pltpu.make_async_copy(p_hbm.at[si.at[b]], r1.at[b], sa[b]).wait()
            pltpu.make_async_copy(q_hbm.at[di.at[b]], r2.at[b], sb[b]).wait()
            pltpu.sync_copy(r1.at[b], g1_hbm.at[pl.ds(off, CH)])
            pltpu.sync_copy(r2.at[b], g2_hbm.at[pl.ds(off, CH)])

        _pipe2(nch, issue, process)

    return k(p, q, src, dst)


AGGP = 10240


def _sc_scatter_add(e, dst, zeros):
    per = E // NTILE
    nch = per // CH
    zper = AGGP // 16

    @functools.partial(
        pl.kernel,
        out_type=jax.ShapeDtypeStruct((2 * AGGP, W), F32),
        mesh=_sc_mesh(),
        scratch_types=[
            pltpu.VMEM((2, CH), jnp.int32),
            pltpu.VMEM((2, CH, W), F32),
            pltpu.VMEM_SHARED((AGGP, W), F32),
            pltpu.SemaphoreType.DMA,
            pltpu.SemaphoreType.DMA,
        ],
    )
    def k(e_hbm, d_hbm, z_hbm, o_hbm, di, rv, agg, s0, s1):
        cid = lax.axis_index("c")
        sid = lax.axis_index("s")
        wid = cid * 16 + sid
        pltpu.sync_copy(
            z_hbm.at[pl.ds(sid * zper, zper)], agg.at[pl.ds(sid * zper, zper)]
        )
        plsc.subcore_barrier()
        base = wid * per
        sems = (s0, s1)

        def issue(c, b):
            off = base + c * CH
            pltpu.sync_copy(d_hbm.at[pl.ds(off, CH)], di.at[b])
            pltpu.async_copy(e_hbm.at[pl.ds(off, CH)], rv.at[b], sems[b])

        def process(c, b):
            off = base + c * CH
            pltpu.make_async_copy(
                e_hbm.at[pl.ds(off, CH)], rv.at[b], sems[b]
            ).wait()
            pltpu.sync_copy(rv.at[b], agg.at[di.at[b]], add=True)

        _pipe2(nch, issue, process)

        plsc.subcore_barrier()
        pltpu.sync_copy(
            agg.at[pl.ds(sid * zper, zper)],
            o_hbm.at[pl.ds(cid * AGGP + sid * zper, zper)],
        )

    return k(e, dst, zeros)


def _r(b):
    return b.reshape(1, W)


def kernel(edge_index, v, c_skip, e_skip, edge_index_skip, batch_skip,
           up_W1, up_b1, up_W2, up_b2,
           eW1, eb1, eW2, eb2, nW1, nb1, nW2, nb2):
    del batch_skip
    src = edge_index[0]
    dst = edge_index[1]
    n_pad = 10240
    eis_pad = jnp.concatenate(
        [edge_index_skip, jnp.zeros((n_pad - N_FINE,), jnp.int32)]
    )
    zeros = jnp.zeros((AGGP, W), F32)

    h = _tc_unpool(v, up_W1, _r(up_b1), up_W2, _r(up_b2))
    vf_pad = _sc_gather1(h, eis_pad, n_pad)
    v_cur, p, q = _tc_nodeprep(
        vf_pad, c_skip, eW1[0, W:2 * W], eW1[0, 2 * W:], _r(eb1[0])
    )

    e = e_skip
    for d in range(2):
        g1, g2 = _sc_gather2(p, q, src, dst)
        e = _tc_edge(e, g1, g2, eW1[d, :W], eW2[d], _r(eb2[d]))
        parts = _sc_scatter_add(e, dst, zeros)
        p0 = parts[:N_FINE]
        p1 = parts[AGGP:AGGP + N_FINE]
        if d == 0:
            v_cur, p, q = _tc_node_mid(
                v_cur, p0, p1, nW1[d, :W], nW1[d, W:], _r(nb1[d]),
                nW2[d], _r(nb2[d]),
                eW1[1, W:2 * W], eW1[1, 2 * W:], _r(eb1[1]),
            )
        else:
            v_cur = _tc_node_fin(
                v_cur, p0, p1, nW1[d, :W], nW1[d, W:], _r(nb1[d]),
                nW2[d], _r(nb2[d]),
            )
    return v_cur
